# baseline (speedup 1.0000x reference)
import functools

import jax
import jax.numpy as jnp
from jax import lax
from jax.experimental import pallas as pl
from jax.experimental.pallas import tpu as pltpu
from jax.experimental.pallas import tpu_sc as plsc

_MARGIN = 1.0
_L = 16

_DNUMS = lax.GatherDimensionNumbers(
    offset_dims=(), collapsed_slice_dims=(0,), start_index_map=(0,))


def _loss_kernel(T, B, D):
    DPT = 4
    TH = T // 2
    NW = 32
    NTR = B // 128
    NJ = TH // 128
    mesh = plsc.VectorSubcoreMesh(core_axis_name="c", subcore_axis_name="s")

    @functools.partial(
        pl.kernel,
        mesh=mesh,
        out_type=jax.ShapeDtypeStruct((NW, _L), jnp.float32),
        compiler_params=pltpu.CompilerParams(
            use_tc_tiling_on_sc=False, needs_layout_passes=False),
        scratch_types=[
            pltpu.VMEM((DPT * NTR, 128), jnp.float32),
            pltpu.VMEM((DPT, NTR), jnp.int32),
            pltpu.VMEM((TH,), jnp.int32),
            pltpu.VMEM((TH,), jnp.int32),
            pltpu.VMEM((TH,), jnp.int32),
            pltpu.VMEM((NJ, 128), jnp.float32),
            pltpu.VMEM((NJ,), jnp.int32),
            pltpu.VMEM((DPT, 128), jnp.float32),
            pltpu.VMEM((_L,), jnp.float32),
            pltpu.VMEM_SHARED((NJ, 128), jnp.float32),
            pltpu.SemaphoreType.DMA,
        ],
    )
    def k(emb_hbm, trip_hbm, out_hbm, rows_v, gidx_v, ia_v, ip_v,
          in_v, part_v, sidx_v, sl_v, out_v, acc_sh, sem):
        c = lax.axis_index("c")
        s = lax.axis_index("s")
        d0 = s * DPT
        tb = c * TH

        lanes = lax.iota(jnp.int32, _L)
        lanes8 = lanes * 8

        trip_copies = [
            pltpu.make_async_copy(trip_hbm.at[pl.ds(r * T + tb, TH)], v, sem)
            for r, v in ((0, ia_v), (1, ip_v), (2, in_v))
        ]
        for cpy in trip_copies:
            cpy.start()

        for i in range(DPT):
            d = d0 + i
            base = (d >> 3) * (8 * NTR) + (d & 7)
            for kk in range(NTR // _L):
                gidx_v[i, pl.ds(kk * _L, _L)] = lanes8 + (base + 128 * kk)
        row_copies = [
            pltpu.make_async_copy(emb_hbm.at[gidx_v.at[i]],
                                  rows_v.at[pl.ds(i * NTR, NTR)], sem)
            for i in range(DPT)
        ]
        for cpy in row_copies:
            cpy.start()

        for kk in range(NJ // _L):
            sidx_v[pl.ds(kk * _L, _L)] = lanes + (kk * _L)

        for cpy in trip_copies:
            cpy.wait()
        for cpy in row_copies:
            cpy.wait()

        def one_group(g):
            j = g >> 3
            co = (g & 7) * _L
            sl = pl.ds(co, _L)
            gsl = pl.ds(g * _L, _L)
            iav = ia_v[gsl]
            ipv = ip_v[gsl]
            inv = in_v[gsl]
            his = [v >> 7 for v in (iav, ipv, inv)]
            los = [v & 127 for v in (iav, ipv, inv)]
            contrib = None
            for i in range(DPT):
                if i == 0:
                    rows = his
                else:
                    rows = [h + (i * NTR) for h in his]
                a = plsc.load_gather(rows_v, [rows[0], los[0]])
                p = plsc.load_gather(rows_v, [rows[1], los[1]])
                n = plsc.load_gather(rows_v, [rows[2], los[2]])
                m = (p - n) * ((p + n) - a - a)
                contrib = m if contrib is None else contrib + m
            part_v[j, sl] = contrib

        UN = 2
        def body(gi, carry):
            for u in range(UN):
                one_group(UN * gi + u)
            return carry

        lax.fori_loop(0, TH // (UN * _L), body, jnp.int32(0))

        @pl.when(s == 0)
        def _():
            pltpu.sync_copy(part_v, acc_sh)
        plsc.subcore_barrier()

        @pl.when(s != 0)
        def _():
            pltpu.sync_copy(part_v, acc_sh.at[sidx_v], add=True)
        plsc.subcore_barrier()

        pltpu.sync_copy(acc_sh.at[pl.ds(s * DPT, DPT)], sl_v)

        perm_idx = {sh: (lanes ^ sh)[:, None] for sh in (1, 2, 4, 8)}

        def permute(v, sh):
            return lax.gather(v, perm_idx[sh], _DNUMS, (1,),
                              mode=lax.GatherScatterMode.PROMISE_IN_BOUNDS)

        acc = jnp.zeros((_L,), jnp.float32)
        for row in range(DPT):
            for kk in range(128 // _L):
                v = sl_v[row, pl.ds(kk * _L, _L)]
                acc = acc + jnp.maximum(v + _MARGIN, 0.0)
        for sh in (8, 4, 2, 1):
            acc = acc + permute(acc, sh)
        out_v[...] = acc
        wid = s * 2 + c
        pltpu.sync_copy(out_v, out_hbm.at[wid])

    return k


def kernel(embeddings, target, triplets):
    del target
    T = triplets.shape[0]
    B, D = embeddings.shape
    emb_tiles = (embeddings.T.reshape(D // 8, 8, B // 128, 128)
                 .transpose(0, 2, 1, 3).reshape((B * D) // 128, 128))
    trip_flat = triplets.T.reshape(-1)
    out = _loss_kernel(T, B, D)(emb_tiles, trip_flat)
    return (jnp.sum(out[:, 0]) / T, T)


# speedup vs baseline: 1.0554x; 1.0554x over previous
import functools

import jax
import jax.numpy as jnp
from jax import lax
from jax.experimental import pallas as pl
from jax.experimental.pallas import tpu as pltpu
from jax.experimental.pallas import tpu_sc as plsc

_MARGIN = 1.0
_L = 16

_DNUMS = lax.GatherDimensionNumbers(
    offset_dims=(), collapsed_slice_dims=(0,), start_index_map=(0,))


def _loss_kernel(T, B, D):
    DPT = 4
    TH = T // 2
    NW = 32
    NTR = B // 128
    NJ = TH // 128
    mesh = plsc.VectorSubcoreMesh(core_axis_name="c", subcore_axis_name="s")

    @functools.partial(
        pl.kernel,
        mesh=mesh,
        out_type=jax.ShapeDtypeStruct((NW, _L), jnp.float32),
        compiler_params=pltpu.CompilerParams(
            use_tc_tiling_on_sc=False, needs_layout_passes=False),
        scratch_types=[
            pltpu.VMEM((DPT * NTR, 128), jnp.float32),
            pltpu.VMEM((DPT, NTR), jnp.int32),
            pltpu.VMEM((3, NJ), jnp.int32),
            pltpu.VMEM((NJ, 128), jnp.int32),
            pltpu.VMEM((NJ, 128), jnp.int32),
            pltpu.VMEM((NJ, 128), jnp.int32),
            pltpu.VMEM((NJ, 128), jnp.float32),
            pltpu.VMEM((NJ,), jnp.int32),
            pltpu.VMEM((DPT, 128), jnp.float32),
            pltpu.VMEM((_L,), jnp.float32),
            pltpu.VMEM_SHARED((NJ, 128), jnp.float32),
            pltpu.SemaphoreType.DMA,
        ],
    )
    def k(emb_hbm, trip_hbm, out_hbm, rows_v, gidx_v, tidx_v, ia_v, ip_v,
          in_v, part_v, sidx_v, sl_v, out_v, acc_sh, sem):
        c = lax.axis_index("c")
        s = lax.axis_index("s")
        d0 = s * DPT

        lanes = lax.iota(jnp.int32, _L)
        lanes4 = lanes * 4
        lanes8 = lanes * 8

        for r in range(3):
            base_r = c * (4 * NJ) + r
            for kk in range(NJ // _L):
                tidx_v[r, pl.ds(kk * _L, _L)] = lanes4 + (base_r + 64 * kk)
        trip_copies = [
            pltpu.make_async_copy(trip_hbm.at[tidx_v.at[r]], v, sem)
            for r, v in ((0, ia_v), (1, ip_v), (2, in_v))
        ]
        for cpy in trip_copies:
            cpy.start()

        for i in range(DPT):
            d = d0 + i
            base = (d >> 3) * (8 * NTR) + (d & 7)
            for kk in range(NTR // _L):
                gidx_v[i, pl.ds(kk * _L, _L)] = lanes8 + (base + 128 * kk)
        row_copies = [
            pltpu.make_async_copy(emb_hbm.at[gidx_v.at[i]],
                                  rows_v.at[pl.ds(i * NTR, NTR)], sem)
            for i in range(DPT)
        ]
        for cpy in row_copies:
            cpy.start()

        for kk in range(NJ // _L):
            sidx_v[pl.ds(kk * _L, _L)] = lanes + (kk * _L)

        for cpy in trip_copies:
            cpy.wait()
        for cpy in row_copies:
            cpy.wait()

        def one_group(g):
            j = g >> 3
            co = (g & 7) * _L
            sl = pl.ds(co, _L)
            iav = ia_v[j, sl]
            ipv = ip_v[j, sl]
            inv = in_v[j, sl]
            his = [v >> 7 for v in (iav, ipv, inv)]
            los = [v & 127 for v in (iav, ipv, inv)]
            contrib = None
            for i in range(DPT):
                if i == 0:
                    rows = his
                else:
                    rows = [h + (i * NTR) for h in his]
                a = plsc.load_gather(rows_v, [rows[0], los[0]])
                p = plsc.load_gather(rows_v, [rows[1], los[1]])
                n = plsc.load_gather(rows_v, [rows[2], los[2]])
                m = (p - n) * ((p + n) - a - a)
                contrib = m if contrib is None else contrib + m
            part_v[j, sl] = contrib

        UN = 2
        def body(gi, carry):
            for u in range(UN):
                one_group(UN * gi + u)
            return carry

        lax.fori_loop(0, TH // (UN * _L), body, jnp.int32(0))

        @pl.when(s == 0)
        def _():
            pltpu.sync_copy(part_v, acc_sh)
        plsc.subcore_barrier()

        @pl.when(s != 0)
        def _():
            pltpu.sync_copy(part_v, acc_sh.at[sidx_v], add=True)
        plsc.subcore_barrier()

        pltpu.sync_copy(acc_sh.at[pl.ds(s * DPT, DPT)], sl_v)

        perm_idx = {sh: (lanes ^ sh)[:, None] for sh in (1, 2, 4, 8)}

        def permute(v, sh):
            return lax.gather(v, perm_idx[sh], _DNUMS, (1,),
                              mode=lax.GatherScatterMode.PROMISE_IN_BOUNDS)

        acc = jnp.zeros((_L,), jnp.float32)
        for row in range(DPT):
            for kk in range(128 // _L):
                v = sl_v[row, pl.ds(kk * _L, _L)]
                acc = acc + jnp.maximum(v + _MARGIN, 0.0)
        for sh in (8, 4, 2, 1):
            acc = acc + permute(acc, sh)
        out_v[...] = acc
        wid = s * 2 + c
        pltpu.sync_copy(out_v, out_hbm.at[wid])

    return k


def kernel(embeddings, target, triplets):
    del target
    T = triplets.shape[0]
    B, D = embeddings.shape
    emb_tiles = (embeddings.T.reshape(D // 8, 8, B // 128, 128)
                 .transpose(0, 2, 1, 3).reshape((B * D) // 128, 128))
    trip_tiles = (jnp.pad(triplets.T, ((0, 1), (0, 0)))
                  .reshape(4, T // 128, 128).transpose(1, 0, 2)
                  .reshape((T * 4) // 128, 128))
    out = _loss_kernel(T, B, D)(emb_tiles, trip_tiles)
    return (jnp.sum(out[:, 0]) / T, T)
